# baseline (speedup 1.0000x reference)
import functools

import jax
import jax.numpy as jnp
from jax import lax
from jax.experimental import pallas as pl
from jax.experimental.pallas import tpu as pltpu
from jax.experimental.pallas import tpu_sc as plsc

_CHUNK = 100
_NBUF = 4


def _sc_embed(idx2, tok, pos, *, nw, nc, nch, chunk, t, d, rows):
    nbuf = _NBUF
    n_grp = nch // nbuf
    lanes = 16
    mesh = plsc.VectorSubcoreMesh(core_axis_name="c", subcore_axis_name="s")

    @functools.partial(
        pl.kernel,
        out_type=jax.ShapeDtypeStruct((rows, d), jnp.float32),
        mesh=mesh,
        compiler_params=pltpu.CompilerParams(use_tc_tiling_on_sc=False),
        scratch_types=[
            pltpu.VMEM((nch, chunk), jnp.int32),
            pltpu.VMEM((t, d), jnp.float32),
            [pltpu.VMEM((chunk, d), jnp.float32) for _ in range(nbuf)],
            [pltpu.SemaphoreType.DMA for _ in range(nbuf)],
            [pltpu.SemaphoreType.DMA for _ in range(nbuf)],
        ],
    )
    def k(idx_hbm, tok_hbm, pos_hbm, out_hbm, idx_v, pos_v, rows_v, gsem, osem):
        wid = lax.axis_index("s") * nc + lax.axis_index("c")
        base_chunk = wid * nch
        base_row = wid * (nch * chunk)

        pltpu.sync_copy(idx_hbm.at[pl.ds(base_chunk, nch)], idx_v)
        pltpu.sync_copy(pos_hbm, pos_v)

        def start_gather(j, b):
            pltpu.async_copy(tok_hbm.at[idx_v.at[j]], rows_v[b], gsem[b])

        def wait_gather(b):
            pltpu.make_async_copy(
                tok_hbm.at[idx_v.at[0]], rows_v[b], gsem[b]).wait()

        def start_out(j, b):
            pltpu.async_copy(
                rows_v[b], out_hbm.at[pl.ds(base_row + j * chunk, chunk)],
                osem[b])

        def wait_out(b):
            pltpu.make_async_copy(
                rows_v[b], out_hbm.at[pl.ds(base_row, chunk)], osem[b]).wait()

        def add_pos(b, phase):
            buf = rows_v[b]

            def body(r, carry):
                for c in range(d // lanes):
                    sl = pl.ds(c * lanes, lanes)
                    plsc.addupdate(buf.at[r, sl], pos_v[phase + r, sl])
                return carry

            lax.fori_loop(0, chunk, body, 0, unroll=10)

        def step(j, b, first=False, last=False):
            nb = (b + nbuf - 1) % nbuf
            if not last:
                if not first:
                    wait_out(nb)
                start_gather(j + nbuf - 1, nb)
            wait_gather(b)
            add_pos(b, chunk * (b % 2))
            start_out(j, b)

        for b in range(nbuf - 1):
            start_gather(b, b)

        for b in range(nbuf):
            step(b, b, first=(b == 0))

        def grp(g, carry):
            for b in range(nbuf):
                step(g * nbuf + b, b)
            return carry

        lax.fori_loop(1, n_grp - 1, grp, 0)

        for b in range(nbuf):
            step((n_grp - 1) * nbuf + b, b, last=(b > 0))

        for b in range(nbuf):
            wait_out(b)

    return k(idx2, tok, pos)


def kernel(x, token_table, pos_table):
    b, t = x.shape
    v, d = token_table.shape
    idx = x.reshape(b * t).astype(jnp.int32)

    info = plsc.get_sparse_core_info()
    nc, ns = info.num_cores, info.num_subcores
    nw = nc * ns
    chunk = _CHUNK
    rows = b * t
    nch = rows // (nw * chunk)
    assert rows == nw * nch * chunk and t == 2 * chunk and d % 16 == 0
    assert nch % _NBUF == 0 and (nch * chunk) % t == 0

    idx2 = idx.reshape(nw * nch, chunk)
    out = _sc_embed(idx2, token_table, pos_table,
                    nw=nw, nc=nc, nch=nch, chunk=chunk, t=t, d=d, rows=rows)
    return out.reshape(b, t, d)


# speedup vs baseline: 1.2527x; 1.2527x over previous
import functools

import jax
import jax.numpy as jnp
from jax import lax
from jax.experimental import pallas as pl
from jax.experimental.pallas import tpu as pltpu
from jax.experimental.pallas import tpu_sc as plsc

_CHUNK = 100
_NBUF = 4


def _sc_embed(idx2, tok, pos, *, nw, nc, nch, chunk, t, d, rows):
    nbuf = _NBUF
    n_grp = nch // nbuf
    lanes = 16
    mesh = plsc.VectorSubcoreMesh(core_axis_name="c", subcore_axis_name="s")

    @functools.partial(
        pl.kernel,
        out_type=jax.ShapeDtypeStruct((rows, d), jnp.float32),
        mesh=mesh,
        compiler_params=pltpu.CompilerParams(use_tc_tiling_on_sc=False),
        scratch_types=[
            pltpu.VMEM((nch, chunk), jnp.int32),
            pltpu.VMEM((t, d), jnp.float32),
            [pltpu.VMEM((chunk, d), jnp.float32) for _ in range(nbuf)],
            [pltpu.SemaphoreType.DMA for _ in range(nbuf)],
            [pltpu.SemaphoreType.DMA for _ in range(nbuf)],
        ],
    )
    def k(idx_hbm, tok_hbm, pos_hbm, out_hbm, idx_v, pos_v, rows_v, gsem, osem):
        wid = lax.axis_index("s") * nc + lax.axis_index("c")
        base_chunk = wid * nch
        base_row = wid * (nch * chunk)

        pltpu.sync_copy(idx_hbm.at[pl.ds(base_chunk, nch)], idx_v)
        pltpu.sync_copy(pos_hbm, pos_v)

        def start_gather(j, b):
            pltpu.async_copy(tok_hbm.at[idx_v.at[j]], rows_v[b], gsem[b])

        def wait_gather(b):
            pltpu.make_async_copy(
                tok_hbm.at[idx_v.at[0]], rows_v[b], gsem[b]).wait()

        def start_out(j, b):
            pltpu.async_copy(
                rows_v[b], out_hbm.at[pl.ds(base_row + j * chunk, chunk)],
                osem[b])

        def wait_out(b):
            pltpu.make_async_copy(
                rows_v[b], out_hbm.at[pl.ds(base_row, chunk)], osem[b]).wait()

        def add_pos(b):
            buf = rows_v[b]
            phase = (b % 2) * chunk

            def body(r, carry):
                for c in range(d // lanes):
                    sl = pl.ds(c * lanes, lanes)
                    buf[r, sl] = buf[r, sl] + pos_v[phase + r, sl]
                return carry

            lax.fori_loop(0, chunk, body, 0, unroll=10)

        def step(j, b, reuse=True, start_next=True):
            wait_gather(b)
            add_pos(b)
            start_out(j, b)
            if start_next:
                nb = (b + nbuf - 1) % nbuf
                if reuse:
                    wait_out(nb)
                start_gather(j + nbuf - 1, nb)

        for b in range(nbuf - 1):
            start_gather(b, b)

        step(0, 0, reuse=False)
        for b in range(1, nbuf):
            step(b, b)

        def grp(g, carry):
            for b in range(nbuf):
                step(g * nbuf + b, b)
            return carry

        lax.fori_loop(1, n_grp - 1, grp, 0)

        for b in range(nbuf):
            step((n_grp - 1) * nbuf + b, b, start_next=(b == 0))

        for b in range(nbuf):
            wait_out(b)

    return k(idx2, tok, pos)


def kernel(x, token_table, pos_table):
    b, t = x.shape
    v, d = token_table.shape
    idx = x.reshape(b * t).astype(jnp.int32)

    info = plsc.get_sparse_core_info()
    nc, ns = info.num_cores, info.num_subcores
    nw = nc * ns
    chunk = _CHUNK
    rows = b * t
    nch = rows // (nw * chunk)
    assert rows == nw * nch * chunk and t == 2 * chunk and d % 16 == 0
    assert nch % _NBUF == 0 and (nch * chunk) % t == 0

    idx2 = idx.reshape(nw * nch, chunk)
    out = _sc_embed(idx2, token_table, pos_table,
                    nw=nw, nc=nc, nch=nch, chunk=chunk, t=t, d=d, rows=rows)
    return out.reshape(b, t, d)
